# baseline (speedup 1.0000x reference)
import functools

import jax
import jax.numpy as jnp
from jax import lax
from jax.experimental import pallas as pl
from jax.experimental.pallas import tpu as pltpu
from jax.experimental.pallas import tpu_sc as plsc

_N_BITS = 8
_Q_MAX = float(2 ** (_N_BITS - 1) - 1)
_Q = 0.99
_GAMMA_MIN = 0.1
_GAMMA_MAX = 10.0

_NW = 32
_CHUNK = 16384
_BINS1 = 65536
_BINS2 = 32768
_CBLK = 256

_MASK31 = 0x7FFFFFFF


def _zero_ref(ref, length):
    z = jnp.zeros((16,), jnp.int32)

    @plsc.parallel_loop(0, length // 16, unroll=8)
    def _(i):
        ref[pl.ds(i * 16, 16)] = z


def _hist_chunk(buf_ref, hist_ref, nvreg, bin_fn):
    ones = jnp.ones((16,), jnp.int32)

    @plsc.parallel_loop(0, nvreg // 128)
    def _(r):
        for u in range(128):
            v = buf_ref[r, pl.ds(u * 16, 16)]
            key = plsc.bitcast(v, jnp.int32) & _MASK31
            b, m = bin_fn(key)
            plsc.addupdate_scatter(hist_ref, [b], ones, mask=m)


def _stream_hist(x_hbm, buf_ref, sem_a, sem_b, hist_ref, row_base, bin_fn):
    crows = _CHUNK // 2048
    nchunk = (1 << 25) // _NW // _CHUNK
    nvreg = _CHUNK // 16

    def copy(i, half, sem):
        return pltpu.make_async_copy(
            x_hbm.at[pl.ds(row_base + i * crows, crows), :],
            buf_ref.at[half],
            sem,
        )

    copy(0, 0, sem_a).start()

    def body(g, _):
        i = g * 2
        copy(i, 0, sem_a).wait()
        copy(i + 1, 1, sem_b).start()
        _hist_chunk(buf_ref.at[0], hist_ref, nvreg, bin_fn)
        copy(i + 1, 1, sem_b).wait()

        @pl.when(i + 2 < nchunk)
        def _():
            copy(i + 2, 0, sem_a).start()

        _hist_chunk(buf_ref.at[1], hist_ref, nvreg, bin_fn)
        return 0

    lax.fori_loop(0, nchunk // 2, body, 0)


def _local_coarse(hist_ref, coarse_ref, nbins):
    io = lax.iota(jnp.int32, 16)

    def cbody(b, _):
        def inner(j, s):
            return s + hist_ref[pl.ds(b * _CBLK + j * 16, 16)]

        s = lax.fori_loop(0, _CBLK // 16, inner, jnp.zeros((16,), jnp.int32))
        plsc.store_scatter(coarse_ref, [jnp.broadcast_to(b, (16,))],
                           jnp.broadcast_to(jnp.sum(s), (16,)), mask=io == 0)
        return 0

    lax.fori_loop(0, nbins // _CBLK, cbody, 0)


def _sum_rows(buf_ref, out_ref, ncols):
    def body(j, _):
        def inner(r, s):
            return s + buf_ref[r, pl.ds(j * 16, 16)]

        out_ref[pl.ds(j * 16, 16)] = lax.fori_loop(
            0, _NW, inner, jnp.zeros((16,), jnp.int32))
        return 0

    lax.fori_loop(0, ncols // 16, body, 0)


def _scan_ref(ref, nvreg, running0, thresh):
    io = lax.iota(jnp.int32, 16)

    def body(j, carry):
        running, found, idx, below = carry
        v = ref[pl.ds(j * 16, 16)]
        cs = running + plsc.cumsum(v)
        tot = jnp.max(cs)
        hit = jnp.logical_and(found == 0, tot >= thresh)
        lane = plsc.all_reduce_ffs(cs >= thresh)
        lane_s = jnp.max(lane)
        below_c = jnp.sum(jnp.where(io == lane, cs - v, 0))
        idx = jnp.where(hit, j * 16 + lane_s, idx)
        below = jnp.where(hit, below_c, below)
        found = jnp.where(tot >= thresh, jnp.int32(1), found)
        return tot, found, idx, below

    _, _, idx, below = lax.fori_loop(
        0, nvreg, body,
        (running0, jnp.int32(0), jnp.int32(0), jnp.int32(0)))
    return idx, below


def _sc_pass1(x_hbm, hist_out, coarse_out, buf, hist, coarse, sem_a, sem_b):
    c = lax.axis_index("c")
    sid = lax.axis_index("s")
    wid = sid * 2 + c
    rows_per_tile = (1 << 25) // 2048 // _NW

    _zero_ref(hist, _BINS1)
    _stream_hist(x_hbm, buf, sem_a, sem_b, hist, wid * rows_per_tile,
                 lambda key: (lax.shift_right_logical(key, 15), None))
    _local_coarse(hist, coarse, _BINS1)
    pltpu.sync_copy(hist, hist_out.at[wid])
    pltpu.sync_copy(coarse, coarse_out.at[wid])


def _find_bucket(hist_hbm, coarse_hbm, cbuf, fbuf, srow, nbins, running0,
                 thresh):
    nblk = nbins // _CBLK
    pltpu.sync_copy(coarse_hbm, cbuf)
    _sum_rows(cbuf, srow, nblk)
    blk, below_blk = _scan_ref(srow, nblk // 16, running0, thresh)
    pltpu.sync_copy(hist_hbm.at[:, pl.ds(blk * _CBLK, _CBLK)], fbuf)
    _sum_rows(fbuf, srow, _CBLK)
    sub, below = _scan_ref(srow, _CBLK // 16, below_blk, thresh)
    return blk * _CBLK + sub, below


def _sc_pass2(x_hbm, hist1, coarse1, hist_out, coarse_out, b1r1_out,
              buf, hist, coarse, cbuf, fbuf, srow, b1v, sem_a, sem_b, *, k1):
    c = lax.axis_index("c")
    sid = lax.axis_index("s")
    wid = sid * 2 + c
    rows_per_tile = (1 << 25) // 2048 // _NW

    b1, below = _find_bucket(hist1, coarse1, cbuf, fbuf, srow, _BINS1,
                             jnp.int32(0), jnp.int32(k1))
    r1 = jnp.int32(k1) - below

    _zero_ref(hist, _BINS2)

    def bin_fn(key):
        return key & jnp.int32(0x7FFF), lax.shift_right_logical(key, 15) == b1

    _stream_hist(x_hbm, buf, sem_a, sem_b, hist, wid * rows_per_tile, bin_fn)
    _local_coarse(hist, coarse, _BINS2)
    pltpu.sync_copy(hist, hist_out.at[wid])
    pltpu.sync_copy(coarse, coarse_out.at[wid])

    @pl.when(jnp.logical_and(c == 0, sid == 0))
    def _():
        io = lax.iota(jnp.int32, 16)
        b1v[...] = jnp.where(io == 0, b1, 0) + jnp.where(io == 1, r1, 0)
        pltpu.sync_copy(b1v, b1r1_out)


def _sc_pass3(hist2, coarse2, b1r1, key_out, cbuf, fbuf, srow, b1v, outv):
    c = lax.axis_index("c")
    sid = lax.axis_index("s")

    @pl.when(jnp.logical_and(c == 0, sid == 0))
    def _():
        pltpu.sync_copy(b1r1, b1v)
        bv = b1v[...]
        b1 = bv[0]
        r1 = bv[1]
        b2, _ = _find_bucket(hist2, coarse2, cbuf, fbuf, srow, _BINS2,
                             jnp.int32(0), r1)
        q_key = b1 * jnp.int32(_BINS2) + b2
        outv[...] = jnp.broadcast_to(q_key, (16,))
        pltpu.sync_copy(outv, key_out)


def _select_key_sc(xi):
    n = xi.size
    k1 = round(_Q * n) + 1
    mesh = plsc.VectorSubcoreMesh(core_axis_name="c", subcore_axis_name="s")
    params = pltpu.CompilerParams(needs_layout_passes=False,
                                  use_tc_tiling_on_sc=True)

    hist1, coarse1 = pl.kernel(
        _sc_pass1,
        out_type=(
            jax.ShapeDtypeStruct((_NW, _BINS1), jnp.int32),
            jax.ShapeDtypeStruct((_NW, _BINS1 // _CBLK), jnp.int32),
        ),
        mesh=mesh,
        compiler_params=params,
        scratch_types=[
            pltpu.VMEM((2, _CHUNK // 2048, 2048), jnp.float32),
            pltpu.VMEM((_BINS1,), jnp.int32),
            pltpu.VMEM((_BINS1 // _CBLK,), jnp.int32),
            pltpu.SemaphoreType.DMA,
            pltpu.SemaphoreType.DMA,
        ],
    )(xi)

    hist2, coarse2, b1r1 = pl.kernel(
        functools.partial(_sc_pass2, k1=k1),
        out_type=(
            jax.ShapeDtypeStruct((_NW, _BINS2), jnp.int32),
            jax.ShapeDtypeStruct((_NW, _BINS2 // _CBLK), jnp.int32),
            jax.ShapeDtypeStruct((16,), jnp.int32),
        ),
        mesh=mesh,
        compiler_params=params,
        scratch_types=[
            pltpu.VMEM((2, _CHUNK // 2048, 2048), jnp.float32),
            pltpu.VMEM((_BINS2,), jnp.int32),
            pltpu.VMEM((_BINS2 // _CBLK,), jnp.int32),
            pltpu.VMEM((_NW, _BINS1 // _CBLK), jnp.int32),
            pltpu.VMEM((_NW, _CBLK), jnp.int32),
            pltpu.VMEM((_BINS1 // _CBLK,), jnp.int32),
            pltpu.VMEM((16,), jnp.int32),
            pltpu.SemaphoreType.DMA,
            pltpu.SemaphoreType.DMA,
        ],
    )(xi, hist1, coarse1)

    key16 = pl.kernel(
        _sc_pass3,
        out_type=jax.ShapeDtypeStruct((16,), jnp.int32),
        mesh=mesh,
        compiler_params=params,
        scratch_types=[
            pltpu.VMEM((_NW, _BINS2 // _CBLK), jnp.int32),
            pltpu.VMEM((_NW, _CBLK), jnp.int32),
            pltpu.VMEM((_BINS2 // _CBLK,), jnp.int32),
            pltpu.VMEM((16,), jnp.int32),
            pltpu.VMEM((16,), jnp.int32),
        ],
    )(hist2, coarse2, b1r1)
    return key16


def _quantize_body(scale_ref, x_ref, out_ref):
    s = scale_ref[0, 0]
    y = x_ref[...] * (1.0 / s)
    y_clip = jnp.clip(y, -_Q_MAX, _Q_MAX)
    y_round = jnp.round(y_clip)
    out_ref[...] = s * (y_clip + (y_round - y_clip))


def kernel(x, gamma):
    n = x.size
    cols = x.shape[-1]
    rows = n // cols
    x2 = x.reshape(rows, cols)

    key16 = _select_key_sc(x2)
    q_quantile = lax.bitcast_convert_type(key16[0], jnp.float32)
    g = jnp.clip(gamma, _GAMMA_MIN, _GAMMA_MAX)
    scale = ((q_quantile / _Q_MAX) * g[0]).reshape(1, 1)

    block_r = 1024
    nchunk = rows // block_r
    out = pl.pallas_call(
        _quantize_body,
        grid=(nchunk,),
        in_specs=[
            pl.BlockSpec((1, 1), lambda c: (0, 0), memory_space=pltpu.SMEM),
            pl.BlockSpec((block_r, cols), lambda c: (c, 0)),
        ],
        out_specs=pl.BlockSpec((block_r, cols), lambda c: (c, 0)),
        out_shape=jax.ShapeDtypeStruct((rows, cols), jnp.float32),
    )(scale, x2)

    return out.reshape(x.shape)


# speedup vs baseline: 2.3917x; 2.3917x over previous
import functools

import jax
import jax.numpy as jnp
from jax import lax
from jax.experimental import pallas as pl
from jax.experimental.pallas import tpu as pltpu
from jax.experimental.pallas import tpu_sc as plsc

_N_BITS = 8
_Q_MAX = float(2 ** (_N_BITS - 1) - 1)
_Q = 0.99
_GAMMA_MIN = 0.1
_GAMMA_MAX = 10.0

_NW = 32
_CHUNK = 16384
_BINS1 = 65536
_BINS2 = 32768
_CBLK = 256

_MASK31 = 0x7FFFFFFF


def _zero_ref(ref, length):
    z = jnp.zeros((16,), jnp.int32)

    @plsc.parallel_loop(0, length // 16, unroll=8)
    def _(i):
        ref[pl.ds(i * 16, 16)] = z


def _hist_chunk(buf_ref, hist_ref, nvreg, bin_fn):
    ones = jnp.ones((16,), jnp.int32)

    @plsc.parallel_loop(0, nvreg, unroll=8)
    def _(i):
        row = lax.shift_right_logical(i, 7)
        col = lax.shift_left(i & 127, 4)
        v = buf_ref[row, pl.ds(col, 16)]
        key = plsc.bitcast(v, jnp.int32) & _MASK31
        b, m = bin_fn(key)
        plsc.addupdate_scatter(hist_ref, [b], ones, mask=m)


def _stream_hist(x_hbm, buf_ref, sem_a, sem_b, hist_ref, row_base, bin_fn):
    crows = _CHUNK // 2048
    nchunk = (1 << 25) // _NW // _CHUNK
    nvreg = _CHUNK // 16

    def copy(i, half, sem):
        return pltpu.make_async_copy(
            x_hbm.at[pl.ds(row_base + i * crows, crows), :],
            buf_ref.at[half],
            sem,
        )

    copy(0, 0, sem_a).start()

    def body(g, _):
        i = g * 2
        copy(i, 0, sem_a).wait()
        copy(i + 1, 1, sem_b).start()
        _hist_chunk(buf_ref.at[0], hist_ref, nvreg, bin_fn)
        copy(i + 1, 1, sem_b).wait()

        @pl.when(i + 2 < nchunk)
        def _():
            copy(i + 2, 0, sem_a).start()

        _hist_chunk(buf_ref.at[1], hist_ref, nvreg, bin_fn)
        return 0

    lax.fori_loop(0, nchunk // 2, body, 0)


def _local_coarse(hist_ref, coarse_ref, nbins):
    io = lax.iota(jnp.int32, 16)

    def cbody(b, _):
        def inner(j, s):
            return s + hist_ref[pl.ds(b * _CBLK + j * 16, 16)]

        s = lax.fori_loop(0, _CBLK // 16, inner, jnp.zeros((16,), jnp.int32))
        plsc.store_scatter(coarse_ref, [jnp.broadcast_to(b, (16,))],
                           jnp.broadcast_to(jnp.sum(s), (16,)), mask=io == 0)
        return 0

    lax.fori_loop(0, nbins // _CBLK, cbody, 0)


def _sum_rows(buf_ref, out_ref, ncols):
    def body(j, _):
        def inner(r, s):
            return s + buf_ref[r, pl.ds(j * 16, 16)]

        out_ref[pl.ds(j * 16, 16)] = lax.fori_loop(
            0, _NW, inner, jnp.zeros((16,), jnp.int32))
        return 0

    lax.fori_loop(0, ncols // 16, body, 0)


def _scan_ref(ref, nvreg, running0, thresh):
    io = lax.iota(jnp.int32, 16)

    def body(j, carry):
        running, found, idx, below = carry
        v = ref[pl.ds(j * 16, 16)]
        cs = running + plsc.cumsum(v)
        tot = jnp.max(cs)
        hit = jnp.logical_and(found == 0, tot >= thresh)
        lane = plsc.all_reduce_ffs(cs >= thresh)
        lane_s = jnp.max(lane)
        below_c = jnp.sum(jnp.where(io == lane, cs - v, 0))
        idx = jnp.where(hit, j * 16 + lane_s, idx)
        below = jnp.where(hit, below_c, below)
        found = jnp.where(tot >= thresh, jnp.int32(1), found)
        return tot, found, idx, below

    _, _, idx, below = lax.fori_loop(
        0, nvreg, body,
        (running0, jnp.int32(0), jnp.int32(0), jnp.int32(0)))
    return idx, below


def _sc_pass1(x_hbm, hist_out, coarse_out, buf, hist, coarse, sem_a, sem_b):
    c = lax.axis_index("c")
    sid = lax.axis_index("s")
    wid = sid * 2 + c
    rows_per_tile = (1 << 25) // 2048 // _NW

    _zero_ref(hist, _BINS1)
    _stream_hist(x_hbm, buf, sem_a, sem_b, hist, wid * rows_per_tile,
                 lambda key: (lax.shift_right_logical(key, 15), None))
    _local_coarse(hist, coarse, _BINS1)
    pltpu.sync_copy(hist, hist_out.at[wid])
    pltpu.sync_copy(coarse, coarse_out.at[wid])


def _find_bucket(hist_hbm, coarse_hbm, cbuf, fbuf, srow, nbins, running0,
                 thresh):
    nblk = nbins // _CBLK
    pltpu.sync_copy(coarse_hbm, cbuf)
    _sum_rows(cbuf, srow, nblk)
    blk, below_blk = _scan_ref(srow, nblk // 16, running0, thresh)
    pltpu.sync_copy(hist_hbm.at[:, pl.ds(blk * _CBLK, _CBLK)], fbuf)
    _sum_rows(fbuf, srow, _CBLK)
    sub, below = _scan_ref(srow, _CBLK // 16, below_blk, thresh)
    return blk * _CBLK + sub, below


def _sc_pass2(x_hbm, hist1, coarse1, hist_out, coarse_out, b1r1_out,
              buf, hist, coarse, cbuf, fbuf, srow, b1v, sem_a, sem_b, *, k1):
    c = lax.axis_index("c")
    sid = lax.axis_index("s")
    wid = sid * 2 + c
    rows_per_tile = (1 << 25) // 2048 // _NW

    b1, below = _find_bucket(hist1, coarse1, cbuf, fbuf, srow, _BINS1,
                             jnp.int32(0), jnp.int32(k1))
    r1 = jnp.int32(k1) - below

    _zero_ref(hist, _BINS2)

    def bin_fn(key):
        return key & jnp.int32(0x7FFF), lax.shift_right_logical(key, 15) == b1

    _stream_hist(x_hbm, buf, sem_a, sem_b, hist, wid * rows_per_tile, bin_fn)
    _local_coarse(hist, coarse, _BINS2)
    pltpu.sync_copy(hist, hist_out.at[wid])
    pltpu.sync_copy(coarse, coarse_out.at[wid])

    @pl.when(jnp.logical_and(c == 0, sid == 0))
    def _():
        io = lax.iota(jnp.int32, 16)
        b1v[...] = jnp.where(io == 0, b1, 0) + jnp.where(io == 1, r1, 0)
        pltpu.sync_copy(b1v, b1r1_out)


def _sc_pass3(hist2, coarse2, b1r1, key_out, cbuf, fbuf, srow, b1v, outv):
    c = lax.axis_index("c")
    sid = lax.axis_index("s")

    @pl.when(jnp.logical_and(c == 0, sid == 0))
    def _():
        pltpu.sync_copy(b1r1, b1v)
        bv = b1v[...]
        b1 = bv[0]
        r1 = bv[1]
        b2, _ = _find_bucket(hist2, coarse2, cbuf, fbuf, srow, _BINS2,
                             jnp.int32(0), r1)
        q_key = b1 * jnp.int32(_BINS2) + b2
        outv[...] = jnp.broadcast_to(q_key, (16,))
        pltpu.sync_copy(outv, key_out)


def _select_key_sc(xi):
    n = xi.size
    k1 = round(_Q * n) + 1
    mesh = plsc.VectorSubcoreMesh(core_axis_name="c", subcore_axis_name="s")
    params = pltpu.CompilerParams(needs_layout_passes=False,
                                  use_tc_tiling_on_sc=True)

    hist1, coarse1 = pl.kernel(
        _sc_pass1,
        out_type=(
            jax.ShapeDtypeStruct((_NW, _BINS1), jnp.int32),
            jax.ShapeDtypeStruct((_NW, _BINS1 // _CBLK), jnp.int32),
        ),
        mesh=mesh,
        compiler_params=params,
        scratch_types=[
            pltpu.VMEM((2, _CHUNK // 2048, 2048), jnp.float32),
            pltpu.VMEM((_BINS1,), jnp.int32),
            pltpu.VMEM((_BINS1 // _CBLK,), jnp.int32),
            pltpu.SemaphoreType.DMA,
            pltpu.SemaphoreType.DMA,
        ],
    )(xi)

    hist2, coarse2, b1r1 = pl.kernel(
        functools.partial(_sc_pass2, k1=k1),
        out_type=(
            jax.ShapeDtypeStruct((_NW, _BINS2), jnp.int32),
            jax.ShapeDtypeStruct((_NW, _BINS2 // _CBLK), jnp.int32),
            jax.ShapeDtypeStruct((16,), jnp.int32),
        ),
        mesh=mesh,
        compiler_params=params,
        scratch_types=[
            pltpu.VMEM((2, _CHUNK // 2048, 2048), jnp.float32),
            pltpu.VMEM((_BINS2,), jnp.int32),
            pltpu.VMEM((_BINS2 // _CBLK,), jnp.int32),
            pltpu.VMEM((_NW, _BINS1 // _CBLK), jnp.int32),
            pltpu.VMEM((_NW, _CBLK), jnp.int32),
            pltpu.VMEM((_BINS1 // _CBLK,), jnp.int32),
            pltpu.VMEM((16,), jnp.int32),
            pltpu.SemaphoreType.DMA,
            pltpu.SemaphoreType.DMA,
        ],
    )(xi, hist1, coarse1)

    key16 = pl.kernel(
        _sc_pass3,
        out_type=jax.ShapeDtypeStruct((16,), jnp.int32),
        mesh=mesh,
        compiler_params=params,
        scratch_types=[
            pltpu.VMEM((_NW, _BINS2 // _CBLK), jnp.int32),
            pltpu.VMEM((_NW, _CBLK), jnp.int32),
            pltpu.VMEM((_BINS2 // _CBLK,), jnp.int32),
            pltpu.VMEM((16,), jnp.int32),
            pltpu.VMEM((16,), jnp.int32),
        ],
    )(hist2, coarse2, b1r1)
    return key16


def _quantize_body(scale_ref, x_ref, out_ref):
    s = scale_ref[0, 0]
    y = x_ref[...] * (1.0 / s)
    y_clip = jnp.clip(y, -_Q_MAX, _Q_MAX)
    y_round = jnp.round(y_clip)
    out_ref[...] = s * (y_clip + (y_round - y_clip))


def kernel(x, gamma):
    n = x.size
    cols = x.shape[-1]
    rows = n // cols
    x2 = x.reshape(rows, cols)

    key16 = _select_key_sc(x2)
    q_quantile = lax.bitcast_convert_type(key16[0], jnp.float32)
    g = jnp.clip(gamma, _GAMMA_MIN, _GAMMA_MAX)
    scale = ((q_quantile / _Q_MAX) * g[0]).reshape(1, 1)

    block_r = 1024
    nchunk = rows // block_r
    out = pl.pallas_call(
        _quantize_body,
        grid=(nchunk,),
        in_specs=[
            pl.BlockSpec((1, 1), lambda c: (0, 0), memory_space=pltpu.SMEM),
            pl.BlockSpec((block_r, cols), lambda c: (c, 0)),
        ],
        out_specs=pl.BlockSpec((block_r, cols), lambda c: (c, 0)),
        out_shape=jax.ShapeDtypeStruct((rows, cols), jnp.float32),
    )(scale, x2)

    return out.reshape(x.shape)
